# initial kernel scaffold (unmeasured)
import jax
import jax.numpy as jnp
from jax import lax
from jax.experimental import pallas as pl
from jax.experimental.pallas import tpu as pltpu


def kernel(
    x,
):
    def body(*refs):
        pass

    out_shape = jax.ShapeDtypeStruct(..., jnp.float32)
    return pl.pallas_call(body, out_shape=out_shape)(...)


# baseline (device time: 81090 ns/iter reference)
import jax
import jax.numpy as jnp
from jax import lax
from jax.experimental import pallas as pl
from jax.experimental.pallas import tpu as pltpu

N_DEV = 4


def kernel(x):
    _, m, n_total = x.shape
    n_out = n_total // N_DEV

    def body(x_ref, out_ref, comm_ref, send_sems, recv_sems):
        my_pos = lax.axis_index("i")
        left = (my_pos - 1 + N_DEV) % N_DEV
        right = (my_pos + 1) % N_DEV

        barrier_sem = pltpu.get_barrier_semaphore()
        for nbr in [left, right]:
            pl.semaphore_signal(
                barrier_sem, inc=1,
                device_id=(nbr,), device_id_type=pl.DeviceIdType.MESH,
            )
        pl.semaphore_wait(barrier_sem, 2)

        c0 = (my_pos - 1 + N_DEV) % N_DEV
        comm_ref[0, :, :] = x_ref[0, :, pl.ds(c0 * n_out, n_out)]

        for s in range(N_DEV - 1):
            send_slot = s % 2
            recv_slot = 1 - send_slot
            rdma = pltpu.make_async_remote_copy(
                src_ref=comm_ref.at[send_slot],
                dst_ref=comm_ref.at[recv_slot],
                send_sem=send_sems.at[s],
                recv_sem=recv_sems.at[s],
                device_id=(right,),
                device_id_type=pl.DeviceIdType.MESH,
            )
            rdma.start()
            rdma.wait()

            c = (my_pos - s - 2 + 2 * N_DEV) % N_DEV
            chunk = x_ref[0, :, pl.ds(c * n_out, n_out)]
            if s < N_DEV - 2:
                comm_ref[recv_slot, :, :] = comm_ref[recv_slot, :, :] + chunk
            else:
                out_ref[:, :] = comm_ref[recv_slot, :, :] + chunk

    return pl.pallas_call(
        body,
        out_shape=jax.ShapeDtypeStruct((m, n_out), x.dtype),
        in_specs=[pl.BlockSpec(memory_space=pltpu.VMEM)],
        out_specs=pl.BlockSpec(memory_space=pltpu.VMEM),
        scratch_shapes=[
            pltpu.VMEM((2, m, n_out), x.dtype),
            pltpu.SemaphoreType.DMA((N_DEV - 1,)),
            pltpu.SemaphoreType.DMA((N_DEV - 1,)),
        ],
        compiler_params=pltpu.CompilerParams(collective_id=0),
    )(x)


# device time: 47710 ns/iter; 1.6996x vs baseline; 1.6996x over previous
import jax
import jax.numpy as jnp
from jax import lax
from jax.experimental import pallas as pl
from jax.experimental.pallas import tpu as pltpu

N_DEV = 4


def kernel(x):
    _, m, n_total = x.shape
    n_out = n_total // N_DEV
    half = n_out // 2

    def body(x_ref, out_ref, ca_ref, cb_ref, sa_send, sa_recv, sb_send, sb_recv):
        my_pos = lax.axis_index("i")
        left = (my_pos - 1 + N_DEV) % N_DEV
        right = (my_pos + 1) % N_DEV

        barrier_sem = pltpu.get_barrier_semaphore()
        for nbr in [left, right]:
            pl.semaphore_signal(
                barrier_sem, inc=1,
                device_id=(nbr,), device_id_type=pl.DeviceIdType.MESH,
            )
        pl.semaphore_wait(barrier_sem, 2)

        ca0 = (my_pos - 1 + N_DEV) % N_DEV
        cb0 = (my_pos + 1) % N_DEV
        ca_ref[0, :, :] = x_ref[0, :, pl.ds(ca0 * n_out, half)]
        cb_ref[0, :, :] = x_ref[0, :, pl.ds(cb0 * n_out + half, half)]

        for s in range(N_DEV - 1):
            ss = s % 2
            rs = 1 - ss
            rdma_a = pltpu.make_async_remote_copy(
                src_ref=ca_ref.at[ss],
                dst_ref=ca_ref.at[rs],
                send_sem=sa_send.at[s],
                recv_sem=sa_recv.at[s],
                device_id=(right,),
                device_id_type=pl.DeviceIdType.MESH,
            )
            rdma_b = pltpu.make_async_remote_copy(
                src_ref=cb_ref.at[ss],
                dst_ref=cb_ref.at[rs],
                send_sem=sb_send.at[s],
                recv_sem=sb_recv.at[s],
                device_id=(left,),
                device_id_type=pl.DeviceIdType.MESH,
            )
            rdma_a.start()
            rdma_b.start()
            rdma_a.wait()
            rdma_b.wait()

            ca = (my_pos - s - 2 + 2 * N_DEV) % N_DEV
            cb = (my_pos + s + 2) % N_DEV
            chunk_a = x_ref[0, :, pl.ds(ca * n_out, half)]
            chunk_b = x_ref[0, :, pl.ds(cb * n_out + half, half)]
            if s < N_DEV - 2:
                ca_ref[rs, :, :] = ca_ref[rs, :, :] + chunk_a
                cb_ref[rs, :, :] = cb_ref[rs, :, :] + chunk_b
            else:
                out_ref[:, :half] = ca_ref[rs, :, :] + chunk_a
                out_ref[:, half:] = cb_ref[rs, :, :] + chunk_b

    return pl.pallas_call(
        body,
        out_shape=jax.ShapeDtypeStruct((m, n_out), x.dtype),
        in_specs=[pl.BlockSpec(memory_space=pltpu.VMEM)],
        out_specs=pl.BlockSpec(memory_space=pltpu.VMEM),
        scratch_shapes=[
            pltpu.VMEM((2, m, half), x.dtype),
            pltpu.VMEM((2, m, half), x.dtype),
            pltpu.SemaphoreType.DMA((N_DEV - 1,)),
            pltpu.SemaphoreType.DMA((N_DEV - 1,)),
            pltpu.SemaphoreType.DMA((N_DEV - 1,)),
            pltpu.SemaphoreType.DMA((N_DEV - 1,)),
        ],
        compiler_params=pltpu.CompilerParams(collective_id=0),
    )(x)


# device time: 43554 ns/iter; 1.8618x vs baseline; 1.0954x over previous
import jax
import jax.numpy as jnp
from jax import lax
from jax.experimental import pallas as pl
from jax.experimental.pallas import tpu as pltpu

N_DEV = 4
S = 2


def kernel(x):
    _, m, n_total = x.shape
    n_out = n_total // N_DEV
    half = n_out // 2
    rows = m // S

    def body(x_ref, out_ref, ca_ref, cb_ref, sa_send, sa_recv, sb_send, sb_recv):
        my_pos = lax.axis_index("i")
        left = (my_pos - 1 + N_DEV) % N_DEV
        right = (my_pos + 1) % N_DEV

        def a_cols(c):
            return pl.ds(c * n_out, half)

        def b_cols(c):
            return pl.ds(c * n_out + half, half)

        def make(comm, send_sems, recv_sems, t, g, tgt, src):
            return pltpu.make_async_remote_copy(
                src_ref=src,
                dst_ref=comm.at[t, g],
                send_sem=send_sems.at[t, g],
                recv_sem=recv_sems.at[t, g],
                device_id=(tgt,),
                device_id_type=pl.DeviceIdType.MESH,
            )

        barrier_sem = pltpu.get_barrier_semaphore()
        for nbr in [left, right]:
            pl.semaphore_signal(
                barrier_sem, inc=1,
                device_id=(nbr,), device_id_type=pl.DeviceIdType.MESH,
            )
        pl.semaphore_wait(barrier_sem, 2)

        ca0 = (my_pos - 1 + N_DEV) % N_DEV
        cb0 = (my_pos + 1) % N_DEV

        sends = []
        for g in range(S):
            rsl = pl.ds(g * rows, rows)
            ra = make(ca_ref, sa_send, sa_recv, 0, g, right,
                      x_ref.at[0, rsl, a_cols(ca0)])
            rb = make(cb_ref, sb_send, sb_recv, 0, g, left,
                      x_ref.at[0, rsl, b_cols(cb0)])
            ra.start()
            rb.start()
            sends += [ra, rb]

        for t in range(1, N_DEV - 1):
            ca = (my_pos - t - 1 + 2 * N_DEV) % N_DEV
            cb = (my_pos + t + 1) % N_DEV
            for g in range(S):
                rsl = pl.ds(g * rows, rows)
                make(ca_ref, sa_send, sa_recv, t - 1, g, right,
                     ca_ref.at[t - 1, g]).wait_recv()
                ca_ref[t - 1, g] = ca_ref[t - 1, g] + x_ref[0, rsl, a_cols(ca)]
                ra = make(ca_ref, sa_send, sa_recv, t, g, right,
                          ca_ref.at[t - 1, g])
                ra.start()

                make(cb_ref, sb_send, sb_recv, t - 1, g, left,
                     cb_ref.at[t - 1, g]).wait_recv()
                cb_ref[t - 1, g] = cb_ref[t - 1, g] + x_ref[0, rsl, b_cols(cb)]
                rb = make(cb_ref, sb_send, sb_recv, t, g, left,
                          cb_ref.at[t - 1, g])
                rb.start()
                sends += [ra, rb]

        tl = N_DEV - 2
        for g in range(S):
            rsl = pl.ds(g * rows, rows)
            make(ca_ref, sa_send, sa_recv, tl, g, right,
                 ca_ref.at[tl, g]).wait_recv()
            out_ref[g * rows:(g + 1) * rows, 0:half] = (
                ca_ref[tl, g] + x_ref[0, rsl, a_cols(my_pos)]
            )
            make(cb_ref, sb_send, sb_recv, tl, g, left,
                 cb_ref.at[tl, g]).wait_recv()
            out_ref[g * rows:(g + 1) * rows, half:n_out] = (
                cb_ref[tl, g] + x_ref[0, rsl, b_cols(my_pos)]
            )

        for r in sends:
            r.wait_send()

    return pl.pallas_call(
        body,
        out_shape=jax.ShapeDtypeStruct((m, n_out), x.dtype),
        in_specs=[pl.BlockSpec(memory_space=pltpu.VMEM)],
        out_specs=pl.BlockSpec(memory_space=pltpu.VMEM),
        scratch_shapes=[
            pltpu.VMEM((N_DEV - 1, S, rows, half), x.dtype),
            pltpu.VMEM((N_DEV - 1, S, rows, half), x.dtype),
            pltpu.SemaphoreType.DMA((N_DEV - 1, S)),
            pltpu.SemaphoreType.DMA((N_DEV - 1, S)),
            pltpu.SemaphoreType.DMA((N_DEV - 1, S)),
            pltpu.SemaphoreType.DMA((N_DEV - 1, S)),
        ],
        compiler_params=pltpu.CompilerParams(collective_id=0),
    )(x)
